# baseline (speedup 1.0000x reference)
import functools

import jax
import jax.numpy as jnp
from jax import lax
from jax.experimental import pallas as pl
from jax.experimental.pallas import tpu as pltpu
from jax.experimental.pallas import tpu_sc as plsc

VOCAB = 1000000
EMBED_DIM = 64
BATCH = 16384
HIST = 50

_NC, _NS = 2, 16
_NW = _NC * _NS
_BW = BATCH // _NW
_CH = 128
_CPW = _BW // _CH
_NU = HIST * _CPW

_mesh = plsc.VectorSubcoreMesh(core_axis_name="c", subcore_axis_name="s")


@functools.partial(
    pl.kernel,
    out_type=jax.ShapeDtypeStruct((HIST, 8, BATCH // _CH, 8, _CH), jnp.float32),
    mesh=_mesh,
    scratch_types=[
        pltpu.VMEM((HIST, _BW), jnp.int32),
        pltpu.VMEM((_CH, EMBED_DIM), jnp.float32),
        pltpu.VMEM((_CH, EMBED_DIM), jnp.float32),
        pltpu.VMEM((8, 8, _CH), jnp.float32),
        pltpu.VMEM((8, 8, _CH), jnp.float32),
        pltpu.SemaphoreType.DMA,
        pltpu.SemaphoreType.DMA,
        pltpu.SemaphoreType.DMA,
        pltpu.SemaphoreType.DMA,
    ],
    compiler_params=pltpu.CompilerParams(
        use_tc_tiling_on_sc=False, needs_layout_passes=False
    ),
)
def _gather_kernel(xt_hbm, table_hbm, out_hbm, idxb, g0, g1, t0, t1,
                   sg0, sg1, ss0, ss1):
    wid = lax.axis_index("s") * _NC + lax.axis_index("c")
    b0 = wid * _BW
    g = (g0, g1)
    t = (t0, t1)
    sg = (sg0, sg1)
    ss = (ss0, ss1)

    pltpu.sync_copy(xt_hbm.at[:, pl.ds(pl.multiple_of(b0, _BW), _BW)], idxb)

    jv16 = lax.iota(jnp.int32, 16)

    def unit_hc(u):
        return u // _CPW, wid * _CPW + lax.rem(u, _CPW)

    def fire_gather(u, b):
        h, _ = unit_hc(u)
        c4 = lax.rem(u, _CPW)
        pltpu.async_copy(
            table_hbm.at[idxb.at[h, pl.ds(c4 * _CH, _CH)]], g[b], sg[b]
        )

    def drain_gather(b):
        pltpu.make_async_copy(table_hbm.at[pl.ds(0, _CH)], g[b], sg[b]).wait()

    def out_slice(u):
        h, c = unit_hc(u)
        return out_hbm.at[h, :, c, :, :]

    def wait_store(u, b):
        pltpu.make_async_copy(t[b], out_slice(u), ss[b]).wait()

    def transpose(b):
        for d in range(EMBED_DIM):
            dv16 = jnp.full((16,), d, dtype=jnp.int32)
            for jg in range(_CH // 16):
                vec = plsc.load_gather(g[b], [jv16 + (jg * 16), dv16])
                t[b][d // 8, d % 8, pl.ds(jg * 16, 16)] = vec

    fire_gather(0, 0)
    fire_gather(1, 1)

    def body(u, b, fire_next, first):
        drain_gather(b)
        if not first:
            wait_store(u - 2, b)
        transpose(b)
        if fire_next:
            fire_gather(u + 2, b)
        pltpu.async_copy(t[b], out_slice(u), ss[b])

    body(0, 0, True, True)
    body(1, 1, True, True)

    def outer(i, carry):
        for b in range(2):
            body(2 * i + b, b, True, False)
        return carry

    lax.fori_loop(1, _NU // 2 - 1, outer, 0)

    body(_NU - 2, 0, False, False)
    body(_NU - 1, 1, False, False)
    wait_store(_NU - 2, 0)
    wait_store(_NU - 1, 1)


def kernel(x, embedding):
    xt = jnp.transpose(x).astype(jnp.int32)
    out5 = _gather_kernel(xt, embedding)
    out = out5.transpose(2, 4, 0, 1, 3).reshape(BATCH, HIST, EMBED_DIM)
    return out


# speedup vs baseline: 1.2341x; 1.2341x over previous
import functools

import jax
import jax.numpy as jnp
from jax import lax
from jax.experimental import pallas as pl
from jax.experimental.pallas import tpu as pltpu
from jax.experimental.pallas import tpu_sc as plsc

VOCAB = 1000000
EMBED_DIM = 64
BATCH = 16384
HIST = 50

_B = BATCH * HIST
_NC, _NS = 2, 16
_NW = _NC * _NS
_BPW = _B // _NW
_CH = 128
_NK = 4
_SUP = _NK * _CH
_NSUP = _BPW // _SUP
_NIR = _BPW // _CH

_mesh = plsc.VectorSubcoreMesh(core_axis_name="c", subcore_axis_name="s")


@functools.partial(
    pl.kernel,
    out_type=jax.ShapeDtypeStruct((_B, EMBED_DIM), jnp.float32),
    mesh=_mesh,
    scratch_types=[
        pltpu.VMEM((_NIR, _CH), jnp.int32),
        pltpu.VMEM((_SUP, EMBED_DIM), jnp.float32),
        pltpu.VMEM((_SUP, EMBED_DIM), jnp.float32),
        pltpu.SemaphoreType.DMA,
        pltpu.SemaphoreType.DMA,
        pltpu.SemaphoreType.DMA,
        pltpu.SemaphoreType.DMA,
    ],
    compiler_params=pltpu.CompilerParams(use_tc_tiling_on_sc=False),
)
def _gather_kernel(idx_hbm, table_hbm, out_hbm, idx_all, rows0, rows1,
                   sg0, sg1, ss0, ss1):
    wid = lax.axis_index("s") * _NC + lax.axis_index("c")
    base = wid * _BPW
    rows = (rows0, rows1)
    sg = (sg0, sg1)
    ss = (ss0, ss1)

    pltpu.sync_copy(
        idx_hbm.at[pl.ds(pl.multiple_of(base // _CH, 8), _NIR)], idx_all
    )

    def out_slice(g):
        return out_hbm.at[pl.ds(pl.multiple_of(base + g * _SUP, _SUP), _SUP)]

    def fire_gathers(g, b):
        for j in range(_NK):
            pltpu.async_copy(
                table_hbm.at[idx_all.at[g * _NK + j]],
                rows[b].at[pl.ds(j * _CH, _CH)],
                sg[b],
            )

    def drain_gathers(b):
        pltpu.make_async_copy(out_slice(0), rows[b], sg[b]).wait()

    def wait_store(g, b):
        pltpu.make_async_copy(rows[b], out_slice(g), ss[b]).wait()

    fire_gathers(0, 0)
    fire_gathers(1, 1)
    drain_gathers(0)
    pltpu.async_copy(rows[0], out_slice(0), ss[0])

    def outer(i, carry):
        for b in range(2):
            g = 2 * i + b
            wait_store(g - 2, b)
            fire_gathers(g, b)
            drain_gathers(1 - b)
            pltpu.async_copy(rows[1 - b], out_slice(g - 1), ss[1 - b])
        return carry

    lax.fori_loop(1, _NSUP // 2, outer, 0)

    wait_store(_NSUP - 2, 0)
    drain_gathers(1)
    pltpu.async_copy(rows[1], out_slice(_NSUP - 1), ss[1])
    wait_store(_NSUP - 1, 1)


def kernel(x, embedding):
    idx = x.reshape(_B // _CH, _CH).astype(jnp.int32)
    out = _gather_kernel(idx, embedding)
    out = lax.optimization_barrier(out)
    return out.reshape(BATCH, HIST, EMBED_DIM)


# speedup vs baseline: 1.6048x; 1.3003x over previous
import functools

import jax
import jax.numpy as jnp
from jax import lax
from jax.experimental import pallas as pl
from jax.experimental.pallas import tpu as pltpu
from jax.experimental.pallas import tpu_sc as plsc

VOCAB = 1000000
EMBED_DIM = 64
BATCH = 16384
HIST = 50

_NC, _NS = 2, 16
_NW = _NC * _NS
_BW = BATCH // _NW
_CH = 128
_CPW = _BW // _CH
_NU = HIST * _CPW

_mesh = plsc.VectorSubcoreMesh(core_axis_name="c", subcore_axis_name="s")


@functools.partial(
    pl.kernel,
    out_type=jax.ShapeDtypeStruct((HIST, 8, BATCH // _CH, 8, _CH), jnp.float32),
    mesh=_mesh,
    scratch_types=[
        pltpu.VMEM((HIST, _BW), jnp.int32),
        pltpu.VMEM((_CH, EMBED_DIM), jnp.float32),
        pltpu.VMEM((_CH, EMBED_DIM), jnp.float32),
        pltpu.VMEM((8, 8, _CH), jnp.float32),
        pltpu.VMEM((8, 8, _CH), jnp.float32),
        pltpu.SemaphoreType.DMA,
        pltpu.SemaphoreType.DMA,
        pltpu.SemaphoreType.DMA,
        pltpu.SemaphoreType.DMA,
    ],
    compiler_params=pltpu.CompilerParams(
        use_tc_tiling_on_sc=False, needs_layout_passes=False
    ),
)
def _gather_kernel(xt_hbm, table_hbm, out_hbm, idxb, g0, g1, t0, t1,
                   sg0, sg1, ss0, ss1):
    wid = lax.axis_index("s") * _NC + lax.axis_index("c")
    b0 = wid * _BW
    g = (g0, g1)
    t = (t0, t1)
    sg = (sg0, sg1)
    ss = (ss0, ss1)

    pltpu.sync_copy(xt_hbm.at[:, pl.ds(pl.multiple_of(b0, _BW), _BW)], idxb)

    jv16 = lax.iota(jnp.int32, 16)
    jvs = [jv16 + (jg * 16) for jg in range(_CH // 16)]

    def unit_hc(u):
        return u // _CPW, wid * _CPW + lax.rem(u, _CPW)

    def fire_gather(u, b):
        h, _ = unit_hc(u)
        c4 = lax.rem(u, _CPW)
        pltpu.async_copy(
            table_hbm.at[idxb.at[h, pl.ds(c4 * _CH, _CH)]], g[b], sg[b]
        )

    def drain_gather(b):
        pltpu.make_async_copy(table_hbm.at[pl.ds(0, _CH)], g[b], sg[b]).wait()

    def out_slice(u):
        h, c = unit_hc(u)
        return out_hbm.at[h, :, c, :, :]

    def wait_store(u, b):
        pltpu.make_async_copy(t[b], out_slice(u), ss[b]).wait()

    def transpose(b):
        @plsc.parallel_loop(0, EMBED_DIM, 1, unroll=2)
        def _(d):
            dv = jnp.full((16,), d, dtype=jnp.int32)
            rv8 = jnp.full((16,), d // 8, dtype=jnp.int32)
            rv = jnp.full((16,), lax.rem(d, 8), dtype=jnp.int32)
            for jg in range(_CH // 16):
                vec = plsc.load_gather(g[b], [jvs[jg], dv])
                plsc.store_scatter(t[b], [rv8, rv, jvs[jg]], vec)

    fire_gather(0, 0)
    fire_gather(1, 1)

    def body(u, b, fire_next, first):
        drain_gather(b)
        if not first:
            wait_store(u - 2, b)
        transpose(b)
        if fire_next:
            fire_gather(u + 2, b)
        pltpu.async_copy(t[b], out_slice(u), ss[b])

    body(0, 0, True, True)
    body(1, 1, True, True)

    def outer(i, carry):
        for b in range(2):
            body(2 * i + b, b, True, False)
        return carry

    lax.fori_loop(1, _NU // 2 - 1, outer, 0)

    body(_NU - 2, 0, False, False)
    body(_NU - 1, 1, False, False)
    wait_store(_NU - 2, 0)
    wait_store(_NU - 1, 1)


def kernel(x, embedding):
    xt = jnp.transpose(x).astype(jnp.int32)
    out5 = _gather_kernel(xt, embedding)
    out = out5.transpose(2, 4, 0, 1, 3).reshape(BATCH, HIST, EMBED_DIM)
    return out


# speedup vs baseline: 1.6109x; 1.0039x over previous
import functools

import jax
import jax.numpy as jnp
from jax import lax
from jax.experimental import pallas as pl
from jax.experimental.pallas import tpu as pltpu
from jax.experimental.pallas import tpu_sc as plsc

VOCAB = 1000000
EMBED_DIM = 64
BATCH = 16384
HIST = 50

_NC, _NS = 2, 16
_NW = _NC * _NS
_BW = BATCH // _NW
_CH = 128
_CPW = _BW // _CH
_NU = HIST * _CPW

_mesh = plsc.VectorSubcoreMesh(core_axis_name="c", subcore_axis_name="s")


@functools.partial(
    pl.kernel,
    out_type=jax.ShapeDtypeStruct((HIST, 8, BATCH // _CH, 8, _CH), jnp.float32),
    mesh=_mesh,
    scratch_types=[
        pltpu.VMEM((HIST, _BW), jnp.int32),
        pltpu.VMEM((_CH, EMBED_DIM), jnp.float32),
        pltpu.VMEM((_CH, EMBED_DIM), jnp.float32),
        pltpu.VMEM((8, 8, _CH), jnp.float32),
        pltpu.VMEM((8, 8, _CH), jnp.float32),
        pltpu.SemaphoreType.DMA,
        pltpu.SemaphoreType.DMA,
        pltpu.SemaphoreType.DMA,
        pltpu.SemaphoreType.DMA,
    ],
    compiler_params=pltpu.CompilerParams(
        use_tc_tiling_on_sc=False, needs_layout_passes=False
    ),
)
def _gather_kernel(xt_hbm, table_hbm, out_hbm, idxb, g0, g1, t0, t1,
                   sg0, sg1, ss0, ss1):
    wid = lax.axis_index("s") * _NC + lax.axis_index("c")
    b0 = wid * _BW
    g = (g0, g1)
    t = (t0, t1)
    sg = (sg0, sg1)
    ss = (ss0, ss1)

    pltpu.sync_copy(xt_hbm.at[:, pl.ds(pl.multiple_of(b0, _BW), _BW)], idxb)

    jv16 = lax.iota(jnp.int32, 16)
    jvs = [jv16 + (jg * 16) for jg in range(_CH // 16)]

    def unit_hc(u):
        return u // _CPW, wid * _CPW + lax.rem(u, _CPW)

    def fire_gather(u, b):
        h, _ = unit_hc(u)
        c4 = lax.rem(u, _CPW)
        pltpu.async_copy(
            table_hbm.at[idxb.at[h, pl.ds(c4 * _CH, _CH)]], g[b], sg[b]
        )

    def drain_gather(b):
        pltpu.make_async_copy(table_hbm.at[pl.ds(0, _CH)], g[b], sg[b]).wait()

    def out_slice(u):
        h, c = unit_hc(u)
        return out_hbm.at[h, :, c, :, :]

    def wait_store(u, b):
        pltpu.make_async_copy(t[b], out_slice(u), ss[b]).wait()

    def transpose(b):
        @plsc.parallel_loop(0, EMBED_DIM, 1, unroll=4)
        def _(d):
            dv = jnp.full((16,), d, dtype=jnp.int32)
            d8 = d // 8
            r8 = lax.rem(d, 8)
            for jg in range(_CH // 16):
                vec = plsc.load_gather(g[b], [jvs[jg], dv])
                t[b][d8, r8, pl.ds(jg * 16, 16)] = vec

    fire_gather(0, 0)
    fire_gather(1, 1)

    def body(u, b, fire_next, first):
        drain_gather(b)
        if not first:
            wait_store(u - 2, b)
        transpose(b)
        if fire_next:
            fire_gather(u + 2, b)
        pltpu.async_copy(t[b], out_slice(u), ss[b])

    body(0, 0, True, True)
    body(1, 1, True, True)

    def outer(i, carry):
        for b in range(2):
            body(2 * i + b, b, True, False)
        return carry

    lax.fori_loop(1, _NU // 2 - 1, outer, 0)

    body(_NU - 2, 0, False, False)
    body(_NU - 1, 1, False, False)
    wait_store(_NU - 2, 0)
    wait_store(_NU - 1, 1)


def kernel(x, embedding):
    xt = jnp.transpose(x).astype(jnp.int32)
    out5 = _gather_kernel(xt, embedding)
    out = out5.transpose(2, 4, 0, 1, 3).reshape(BATCH, HIST, EMBED_DIM)
    return out


# speedup vs baseline: 2.5508x; 1.5834x over previous
import functools

import jax
import jax.numpy as jnp
from jax import lax
from jax.experimental import pallas as pl
from jax.experimental.pallas import tpu as pltpu
from jax.experimental.pallas import tpu_sc as plsc

VOCAB = 1000000
EMBED_DIM = 64
BATCH = 16384
HIST = 50

_NC, _NS = 2, 16
_NW = _NC * _NS
_BW = BATCH // _NW
_CH = 128
_CPW = _BW // _CH
_NU = HIST * _CPW
_TP = _CH + 8

_mesh = plsc.VectorSubcoreMesh(core_axis_name="c", subcore_axis_name="s")


@functools.partial(
    pl.kernel,
    out_type=jax.ShapeDtypeStruct((HIST, 8, BATCH // _CH, 8, _CH), jnp.float32),
    mesh=_mesh,
    scratch_types=[
        pltpu.VMEM((HIST, _BW), jnp.int32),
        pltpu.VMEM((_CH, EMBED_DIM), jnp.float32),
        pltpu.VMEM((_CH, EMBED_DIM), jnp.float32),
        pltpu.VMEM((8, 8, _TP), jnp.float32),
        pltpu.VMEM((8, 8, _TP), jnp.float32),
        pltpu.SemaphoreType.DMA,
        pltpu.SemaphoreType.DMA,
        pltpu.SemaphoreType.DMA,
        pltpu.SemaphoreType.DMA,
    ],
    compiler_params=pltpu.CompilerParams(
        use_tc_tiling_on_sc=False, needs_layout_passes=False
    ),
)
def _gather_kernel(xt_hbm, table_hbm, out_hbm, idxb, g0, g1, t0, t1,
                   sg0, sg1, ss0, ss1):
    wid = lax.axis_index("s") * _NC + lax.axis_index("c")
    b0 = wid * _BW
    g = (g0, g1)
    t = (t0, t1)
    sg = (sg0, sg1)
    ss = (ss0, ss1)

    pltpu.sync_copy(xt_hbm.at[:, pl.ds(pl.multiple_of(b0, _BW), _BW)], idxb)

    jv16 = lax.iota(jnp.int32, 16)
    jvs = [jv16 + (jg * 16) for jg in range(_CH // 16)]

    def unit_hc(u):
        return u // _CPW, wid * _CPW + lax.rem(u, _CPW)

    def fire_gather(u, b):
        h, _ = unit_hc(u)
        c4 = lax.rem(u, _CPW)
        pltpu.async_copy(
            table_hbm.at[idxb.at[h, pl.ds(c4 * _CH, _CH)]], g[b], sg[b]
        )

    def drain_gather(b):
        pltpu.make_async_copy(table_hbm.at[pl.ds(0, _CH)], g[b], sg[b]).wait()

    def out_slice(u):
        h, c = unit_hc(u)
        return out_hbm.at[h, :, c, :, :]

    def t_src(b):
        return t[b].at[:, :, pl.ds(0, _CH)]

    def wait_store(u, b):
        pltpu.make_async_copy(t_src(b), out_slice(u), ss[b]).wait()

    dskew = jv16 * 9

    def transpose(b):
        @plsc.parallel_loop(0, EMBED_DIM, 1, unroll=4)
        def _(d0):
            dv = lax.bitwise_and(dskew + d0, jnp.int32(EMBED_DIM - 1))
            rv8 = lax.shift_right_logical(dv, 3)
            rv = lax.bitwise_and(dv, jnp.int32(7))
            for jg in range(_CH // 16):
                vec = plsc.load_gather(g[b], [jvs[jg], dv])
                plsc.store_scatter(t[b], [rv8, rv, jvs[jg]], vec)

    fire_gather(0, 0)
    fire_gather(1, 1)

    def body(u, b, fire_next, first):
        drain_gather(b)
        if not first:
            wait_store(u - 2, b)
        transpose(b)
        if fire_next:
            fire_gather(u + 2, b)
        pltpu.async_copy(t_src(b), out_slice(u), ss[b])

    body(0, 0, True, True)
    body(1, 1, True, True)

    def outer(i, carry):
        for b in range(2):
            body(2 * i + b, b, True, False)
        return carry

    lax.fori_loop(1, _NU // 2 - 1, outer, 0)

    body(_NU - 2, 0, False, False)
    body(_NU - 1, 1, False, False)
    wait_store(_NU - 2, 0)
    wait_store(_NU - 1, 1)


def kernel(x, embedding):
    xt = jnp.transpose(x).astype(jnp.int32)
    out5 = _gather_kernel(xt, embedding)
    out = out5.transpose(2, 4, 0, 1, 3).reshape(BATCH, HIST, EMBED_DIM)
    return out
